# Initial kernel scaffold; baseline (speedup 1.0000x reference)
import jax
import jax.numpy as jnp
from jax.experimental import pallas as pl


def kernel(x, edge_index, W1, b1, W2, b2):
    raise NotImplementedError("write your pallas kernel here")


# speedup vs baseline: 16.0351x; 16.0351x over previous
import functools

import jax
import jax.numpy as jnp
from jax import lax
from jax.experimental import pallas as pl
from jax.experimental.pallas import tpu as pltpu
from jax.experimental.pallas import tpu_sc as plsc

N = 10000
D = 128
C = 16
E = 320000
NC = 2
NS = 16
NW = NC * NS
EDGES_PER_TILE = E // NW
CHUNK = 80
N_CHUNKS = EDGES_PER_TILE // CHUNK
NP = 10240
ROWS_PER_TILE = NP // NS
ZROWS = 128

_MESH = plsc.VectorSubcoreMesh(core_axis_name="c", subcore_axis_name="s")
_SC_PARAMS = pltpu.CompilerParams(use_tc_tiling_on_sc=False)


def _zero_fill(buf, rows, width):
    @pl.loop(0, rows)
    def _(r):
        @pl.loop(0, width // 16)
        def _(j):
            buf[r, pl.ds(j * 16, 16)] = jnp.zeros((16,), jnp.float32)


def _make_sc_agg(width, stage_in_spmem):
    scratch = [
        pltpu.VMEM((CHUNK,), jnp.int32),
        pltpu.VMEM((CHUNK,), jnp.int32),
        pltpu.VMEM((CHUNK, width), jnp.float32),
        pltpu.VMEM((ZROWS, width), jnp.float32),
        pltpu.VMEM_SHARED((NP, width), jnp.float32),
    ]
    if stage_in_spmem:
        scratch.append(pltpu.VMEM_SHARED((N, width), jnp.float32))

    @functools.partial(
        pl.kernel,
        out_type=jax.ShapeDtypeStruct((NC, NP, width), jnp.float32),
        mesh=_MESH,
        scratch_types=scratch,
        compiler_params=_SC_PARAMS,
    )
    def agg(g_hbm, src_hbm, dst_hbm, out_hbm, src_v, dst_v, rows_v, zbuf,
            acc_sh, *rest):
        c = lax.axis_index("c")
        s = lax.axis_index("s")
        wid = c * NS + s
        row0 = s * ROWS_PER_TILE

        if stage_in_spmem:
            g_tab = rest[0]

            @pl.when(s == 0)
            def _():
                pltpu.sync_copy(g_hbm, g_tab)
        else:
            g_tab = g_hbm

        _zero_fill(zbuf, ZROWS, width)

        @pl.loop(0, ROWS_PER_TILE // ZROWS)
        def _(k):
            pltpu.sync_copy(zbuf, acc_sh.at[pl.ds(row0 + k * ZROWS, ZROWS)])

        plsc.subcore_barrier()

        base0 = wid * EDGES_PER_TILE

        @pl.loop(0, N_CHUNKS)
        def _(i):
            base = base0 + i * CHUNK
            pltpu.sync_copy(src_hbm.at[pl.ds(base, CHUNK)], src_v)
            pltpu.sync_copy(dst_hbm.at[pl.ds(base, CHUNK)], dst_v)
            pltpu.sync_copy(g_tab.at[src_v], rows_v)
            pltpu.sync_copy(rows_v, acc_sh.at[dst_v], add=True)

        plsc.subcore_barrier()
        pltpu.sync_copy(acc_sh.at[pl.ds(row0, ROWS_PER_TILE)],
                        out_hbm.at[c].at[pl.ds(row0, ROWS_PER_TILE)])

    return agg


_sc_agg128 = _make_sc_agg(D, stage_in_spmem=False)
_sc_agg16 = _make_sc_agg(C, stage_in_spmem=True)


@functools.partial(
    pl.kernel,
    out_type=jax.ShapeDtypeStruct((NC, NP, 16), jnp.float32),
    mesh=_MESH,
    scratch_types=[
        pltpu.VMEM((CHUNK,), jnp.int32),
        pltpu.VMEM((CHUNK, 16), jnp.float32),
        pltpu.VMEM((ZROWS, 16), jnp.float32),
        pltpu.VMEM_SHARED((NP, 16), jnp.float32),
    ],
    compiler_params=_SC_PARAMS,
)
def _sc_deg(dst_hbm, out_hbm, dst_v, ones_v, zbuf, acc_sh):
    c = lax.axis_index("c")
    s = lax.axis_index("s")
    wid = c * NS + s
    row0 = s * ROWS_PER_TILE

    _zero_fill(zbuf, ZROWS, 16)

    @pl.loop(0, CHUNK)
    def _(r):
        ones_v[r, pl.ds(0, 16)] = jnp.ones((16,), jnp.float32)

    @pl.loop(0, ROWS_PER_TILE // ZROWS)
    def _(k):
        pltpu.sync_copy(zbuf, acc_sh.at[pl.ds(row0 + k * ZROWS, ZROWS)])

    plsc.subcore_barrier()

    base0 = wid * EDGES_PER_TILE

    @pl.loop(0, N_CHUNKS)
    def _(i):
        base = base0 + i * CHUNK
        pltpu.sync_copy(dst_hbm.at[pl.ds(base, CHUNK)], dst_v)
        pltpu.sync_copy(ones_v, acc_sh.at[dst_v], add=True)

    plsc.subcore_barrier()
    pltpu.sync_copy(acc_sh.at[pl.ds(row0, ROWS_PER_TILE)],
                    out_hbm.at[c].at[pl.ds(row0, ROWS_PER_TILE)])


def _dinv_from(dp_ref):
    deg = dp_ref[0, 0:N, 0:1] + dp_ref[1, 0:N, 0:1] + 1.0
    return lax.rsqrt(deg)


def _tc1_body(x_ref, w_ref, dp_ref, o_ref):
    dinv = _dinv_from(dp_ref)
    h = jnp.dot(x_ref[...], w_ref[...], preferred_element_type=jnp.float32)
    o_ref[...] = h * dinv


def _tc2_body(p_ref, g_ref, dp_ref, w_ref, b_ref, o_ref):
    dinv = _dinv_from(dp_ref)
    h1 = dinv * (p_ref[0, 0:N] + p_ref[1, 0:N] + g_ref[...]) + b_ref[...]
    h1 = jnp.maximum(h1, 0.0)
    z = jnp.dot(h1, w_ref[...], preferred_element_type=jnp.float32)
    o_ref[...] = z * dinv


def _tc3_body(p_ref, g_ref, dp_ref, b_ref, o_ref):
    dinv = _dinv_from(dp_ref)
    logits = dinv * (p_ref[0, 0:N] + p_ref[1, 0:N] + g_ref[...]) + b_ref[...]
    m = jnp.max(logits, axis=-1, keepdims=True)
    e = jnp.exp(logits - m)
    o_ref[...] = e / jnp.sum(e, axis=-1, keepdims=True)


def kernel(x, edge_index, W1, b1, W2, b2):
    ei = edge_index.astype(jnp.int32)
    src = ei[0]
    dst = ei[1]

    degp = _sc_deg(dst)

    g1 = pl.pallas_call(
        _tc1_body,
        out_shape=jax.ShapeDtypeStruct((N, D), jnp.float32),
    )(x, W1, degp)

    p1 = _sc_agg128(g1, src, dst)

    g2 = pl.pallas_call(
        _tc2_body,
        out_shape=jax.ShapeDtypeStruct((N, C), jnp.float32),
    )(p1, g1, degp, W2, b1.reshape(1, D))

    p2 = _sc_agg16(g2, src, dst)

    out = pl.pallas_call(
        _tc3_body,
        out_shape=jax.ShapeDtypeStruct((N, C), jnp.float32),
    )(p2, g2, degp, b2.reshape(1, C))

    return out


# speedup vs baseline: 38.6062x; 2.4076x over previous
import functools

import jax
import jax.numpy as jnp
from jax import lax
from jax.experimental import pallas as pl
from jax.experimental.pallas import tpu as pltpu
from jax.experimental.pallas import tpu_sc as plsc

N = 10000
D = 128
C = 16
E = 320000
NC = 2
NS = 16
NW = NC * NS
EDGES_PER_TILE = E // NW
CHUNK = 80
N_CHUNKS = EDGES_PER_TILE // CHUNK
U = 2
NP = 10240
ROWS_PER_TILE = NP // NS
ZROWS = 16

_MESH = plsc.VectorSubcoreMesh(core_axis_name="c", subcore_axis_name="s")
_SC_PARAMS = pltpu.CompilerParams(use_tc_tiling_on_sc=False)


def _zero_fill(buf, rows, width):
    @pl.loop(0, rows)
    def _(r):
        @pl.loop(0, width // 16)
        def _(j):
            buf[r, pl.ds(j * 16, 16)] = jnp.zeros((16,), jnp.float32)


def _zero_acc_slice(zbuf, acc_sh, row0):
    @pl.loop(0, ROWS_PER_TILE // ZROWS)
    def _(k):
        pltpu.sync_copy(zbuf, acc_sh.at[pl.ds(row0 + k * ZROWS, ZROWS)])


def _make_sc_agg(width, stage_in_spmem):
    scratch = [
        pltpu.VMEM((N_CHUNKS, CHUNK), jnp.int32),
        pltpu.VMEM((N_CHUNKS, CHUNK), jnp.int32),
        [pltpu.VMEM((CHUNK, width), jnp.float32) for _ in range(U)],
        pltpu.VMEM((ZROWS, width), jnp.float32),
        pltpu.VMEM_SHARED((NP, width), jnp.float32),
        pltpu.SemaphoreType.DMA((U,)),
        pltpu.SemaphoreType.DMA((U,)),
    ]
    if stage_in_spmem:
        scratch.append(pltpu.VMEM_SHARED((N, width), jnp.float32))

    @functools.partial(
        pl.kernel,
        out_type=jax.ShapeDtypeStruct((NC, NP, width), jnp.float32),
        mesh=_MESH,
        scratch_types=scratch,
        compiler_params=_SC_PARAMS,
    )
    def agg(g_hbm, src_hbm, dst_hbm, out_hbm, src_all, dst_all, rows, zbuf,
            acc_sh, sem_g, sem_s, *rest):
        c = lax.axis_index("c")
        s = lax.axis_index("s")
        wid = c * NS + s
        row0 = s * ROWS_PER_TILE

        if stage_in_spmem:
            g_tab = rest[0]

            @pl.when(s == 0)
            def _():
                pltpu.sync_copy(g_hbm, g_tab)
        else:
            g_tab = g_hbm

        blk0 = wid * N_CHUNKS
        pltpu.sync_copy(src_hbm.at[pl.ds(blk0, N_CHUNKS)], src_all)
        pltpu.sync_copy(dst_hbm.at[pl.ds(blk0, N_CHUNKS)], dst_all)

        _zero_fill(zbuf, ZROWS, width)
        _zero_acc_slice(zbuf, acc_sh, row0)
        plsc.subcore_barrier()

        def g_desc(i, b):
            return pltpu.make_async_copy(g_tab.at[src_all.at[i]], rows[b],
                                         sem_g.at[b])

        def s_desc(i, b):
            return pltpu.make_async_copy(rows[b], acc_sh.at[dst_all.at[i]],
                                         sem_s.at[b])

        @pl.loop(0, (N_CHUNKS - 1) // U)
        def _(it):
            i0 = it * U
            for b in range(U):
                i = i0 + b

                @pl.when(i >= U)
                def _():
                    s_desc(i - U, b).wait()

                g_desc(i, b).start()

                @pl.when(i >= 1)
                def _():
                    g_desc(i - 1, 1 - b).wait()
                    s_desc(i - 1, 1 - b).start(add=True)

        kl = N_CHUNKS - 1
        s_desc(kl - 2, 0).wait()
        g_desc(kl, 0).start()
        g_desc(kl - 1, 1).wait()
        s_desc(kl - 1, 1).start(add=True)
        g_desc(kl, 0).wait()
        s_desc(kl, 0).start(add=True)
        s_desc(kl - 1, 1).wait()
        s_desc(kl, 0).wait()

        plsc.subcore_barrier()
        pltpu.sync_copy(acc_sh.at[pl.ds(row0, ROWS_PER_TILE)],
                        out_hbm.at[c].at[pl.ds(row0, ROWS_PER_TILE)])

    return agg


_sc_agg128 = _make_sc_agg(D, stage_in_spmem=False)
_sc_agg16 = _make_sc_agg(C, stage_in_spmem=True)


@functools.partial(
    pl.kernel,
    out_type=jax.ShapeDtypeStruct((NC, NP, 16), jnp.float32),
    mesh=_MESH,
    scratch_types=[
        pltpu.VMEM((N_CHUNKS, CHUNK), jnp.int32),
        pltpu.VMEM((CHUNK, 16), jnp.float32),
        pltpu.VMEM((ZROWS, 16), jnp.float32),
        pltpu.VMEM_SHARED((NP, 16), jnp.float32),
        pltpu.SemaphoreType.DMA((U,)),
    ],
    compiler_params=_SC_PARAMS,
)
def _sc_deg(dst_hbm, out_hbm, dst_all, ones_v, zbuf, acc_sh, sem_s):
    c = lax.axis_index("c")
    s = lax.axis_index("s")
    wid = c * NS + s
    row0 = s * ROWS_PER_TILE

    pltpu.sync_copy(dst_hbm.at[pl.ds(wid * N_CHUNKS, N_CHUNKS)], dst_all)

    _zero_fill(zbuf, ZROWS, 16)

    @pl.loop(0, CHUNK)
    def _(r):
        ones_v[r, pl.ds(0, 16)] = jnp.ones((16,), jnp.float32)

    _zero_acc_slice(zbuf, acc_sh, row0)
    plsc.subcore_barrier()

    def s_desc(i, b):
        return pltpu.make_async_copy(ones_v, acc_sh.at[dst_all.at[i]],
                                     sem_s.at[b])

    @pl.loop(0, (N_CHUNKS - 1) // U)
    def _(it):
        i0 = it * U
        for b in range(U):
            i = i0 + b

            @pl.when(i >= U)
            def _():
                s_desc(i - U, b).wait()

            s_desc(i, b).start(add=True)

    kl = N_CHUNKS - 1
    s_desc(kl - 2, 0).wait()
    s_desc(kl, 0).start(add=True)
    s_desc(kl - 1, 1).wait()
    s_desc(kl, 0).wait()

    plsc.subcore_barrier()
    pltpu.sync_copy(acc_sh.at[pl.ds(row0, ROWS_PER_TILE)],
                    out_hbm.at[c].at[pl.ds(row0, ROWS_PER_TILE)])


def _dinv_from(dp_ref):
    deg = dp_ref[0, 0:N, 0:1] + dp_ref[1, 0:N, 0:1] + 1.0
    return lax.rsqrt(deg)


def _tc1_body(x_ref, w_ref, dp_ref, o_ref):
    dinv = _dinv_from(dp_ref)
    h = jnp.dot(x_ref[...], w_ref[...], preferred_element_type=jnp.float32)
    o_ref[...] = h * dinv


def _tc2_body(p_ref, g_ref, dp_ref, w_ref, b_ref, o_ref):
    dinv = _dinv_from(dp_ref)
    h1 = dinv * (p_ref[0, 0:N] + p_ref[1, 0:N] + g_ref[...]) + b_ref[...]
    h1 = jnp.maximum(h1, 0.0)
    z = jnp.dot(h1, w_ref[...], preferred_element_type=jnp.float32)
    o_ref[...] = z * dinv


def _tc3_body(p_ref, g_ref, dp_ref, b_ref, o_ref):
    dinv = _dinv_from(dp_ref)
    logits = dinv * (p_ref[0, 0:N] + p_ref[1, 0:N] + g_ref[...]) + b_ref[...]
    m = jnp.max(logits, axis=-1, keepdims=True)
    e = jnp.exp(logits - m)
    o_ref[...] = e / jnp.sum(e, axis=-1, keepdims=True)


def kernel(x, edge_index, W1, b1, W2, b2):
    ei = edge_index.astype(jnp.int32)
    src2 = ei[0].reshape(E // CHUNK, CHUNK)
    dst2 = ei[1].reshape(E // CHUNK, CHUNK)

    degp = _sc_deg(dst2)

    g1 = pl.pallas_call(
        _tc1_body,
        out_shape=jax.ShapeDtypeStruct((N, D), jnp.float32),
    )(x, W1, degp)

    p1 = _sc_agg128(g1, src2, dst2)

    g2 = pl.pallas_call(
        _tc2_body,
        out_shape=jax.ShapeDtypeStruct((N, C), jnp.float32),
    )(p1, g1, degp, W2, b1.reshape(1, D))

    p2 = _sc_agg16(g2, src2, dst2)

    out = pl.pallas_call(
        _tc3_body,
        out_shape=jax.ShapeDtypeStruct((N, C), jnp.float32),
    )(p2, g2, degp, b2.reshape(1, C))

    return out
